# baseline (speedup 1.0000x reference)
import functools

import jax
import jax.numpy as jnp
import numpy as np
from jax import lax
from jax.experimental import pallas as pl
from jax.experimental.pallas import tpu as pltpu
from jax.experimental.pallas import tpu_sc as plsc

_N = 10000
_E = 320000
_DX = 128
_DE = 16
_H = 128
_NG = 16
_L = 16
_NC = 1
_NS = 16
_NW = _NC * _NS
_EPT = _E // _NW
_C = 40
_NCH = _EPT // _C
_BN = 400
_CR = 25
_CW = 512
_SCH = 5
_NSC = _NCH // _SCH
_NZB = _N // _C


_EB = 12800
_NXB = _N // (_E // _EB)


def _bfpack(res):
    lo = lax.bitcast_convert_type(
        res[:, :_H // 2].astype(jnp.bfloat16), jnp.uint16).astype(jnp.uint32)
    hi = lax.bitcast_convert_type(
        res[:, _H // 2:].astype(jnp.bfloat16), jnp.uint16).astype(jnp.uint32)
    return lax.bitcast_convert_type(lo | (hi << 16), jnp.int32)


def _pre_body(a_ref, wa_ref, x_ref, wx_ref, b_ref, oe_ref, ox_ref):
    oe_ref[...] = _bfpack(jnp.dot(a_ref[...], wa_ref[...],
                                  preferred_element_type=jnp.float32))
    ox_ref[...] = (jnp.dot(x_ref[...], wx_ref[...],
                           preferred_element_type=jnp.float32)
                   + b_ref[...])


_pre_call = pl.pallas_call(
    _pre_body,
    grid=(_E // _EB,),
    in_specs=[
        pl.BlockSpec((_EB, _DE), lambda i: (i, 0)),
        pl.BlockSpec((_DE, _H), lambda i: (0, 0)),
        pl.BlockSpec((_NXB, _DX), lambda i: (i, 0)),
        pl.BlockSpec((_DX, _H), lambda i: (0, 0)),
        pl.BlockSpec((1, _H), lambda i: (0, 0)),
    ],
    out_specs=[
        pl.BlockSpec((_EB, _H // 2), lambda i: (i, 0)),
        pl.BlockSpec((_NXB, _H), lambda i: (i, 0)),
    ],
    out_shape=[
        jax.ShapeDtypeStruct((_E, _H // 2), jnp.int32),
        jax.ShapeDtypeStruct((_N, _H), jnp.float32),
    ],
)


@functools.partial(
    pl.kernel,
    out_type=(
        jax.ShapeDtypeStruct((_NC, _N, _H), jnp.float32),
        jax.ShapeDtypeStruct((_NW, _CR, _CW), jnp.int32),
    ),
    mesh=plsc.VectorSubcoreMesh(core_axis_name="c", subcore_axis_name="s",
                                num_cores=_NC, num_subcores=_NS),
    scratch_types=[
        pltpu.VMEM((_SCH, _C), jnp.int32),
        pltpu.VMEM((_SCH, _C), jnp.int32),
        pltpu.VMEM((_SCH * _C + _L,), jnp.int32),
        pltpu.VMEM((_C, _H), jnp.float32),
        pltpu.VMEM((_C, _H), jnp.float32),
        pltpu.VMEM((_C, _H // 2), jnp.int32),
        pltpu.VMEM((_C, _H // 2), jnp.int32),
        pltpu.VMEM((_C, _H), jnp.float32),
        pltpu.VMEM((_C, _H), jnp.float32),
        pltpu.VMEM((_CR, _CW), jnp.int32),
        pltpu.VMEM_SHARED((_N, _H), jnp.float32),
        pltpu.SemaphoreType.DMA,
        pltpu.SemaphoreType.DMA,
        pltpu.SemaphoreType.DMA,
        pltpu.SemaphoreType.DMA,
        pltpu.SemaphoreType.DMA,
        pltpu.SemaphoreType.DMA,
        pltpu.SemaphoreType.DMA,
        pltpu.SemaphoreType.DMA,
    ],
)
def _sc_edge_call(xa_hbm, ea_hbm, row_hbm, col_hbm, cidx_hbm,
                  outs_hbm, outc_hbm, row_v, col_v, cidx_v,
                  g_b0, g_b1, a_b0, a_b1, e_o0, e_o1, hist_v, seg_sh,
                  sem_g0, sem_g1, sem_a0, sem_a1, sem_a2,
                  sem_s0, sem_s1, sem_s2):
    c = lax.axis_index("c")
    s = lax.axis_index("s")
    wid = c * _NS + s

    def _zhrow(r, _):
        for k in range(_CW // _L):
            hist_v[r, pl.ds(k * _L, _L)] = jnp.zeros((_L,), jnp.int32)
        return 0
    lax.fori_loop(0, _CR, _zhrow, 0)

    def _zgrow(r, _):
        for k in range(_H // _L):
            e_o0[r, pl.ds(k * _L, _L)] = jnp.zeros((_L,), jnp.float32)
        return 0
    lax.fori_loop(0, _C, _zgrow, 0)

    iota = lax.iota(jnp.int32, _L)

    for k in range(16):
        idx = s + _NS * k

        @pl.when(idx < _NZB)
        def _():
            off = pl.multiple_of(idx * _C, 8)
            pltpu.sync_copy(e_o0, seg_sh.at[pl.ds(off, _C)])

    plsc.subcore_barrier()

    ebase = pl.multiple_of(wid * _EPT, 8)

    g_bufs = (g_b0, g_b1)
    a_bufs = (a_b0, a_b1)
    e_bufs = (e_o0, e_o1)
    g_sems = (sem_g0, sem_g1)
    a_sems = (sem_a0, sem_a1)
    s_sems = (sem_s0, sem_s1)

    def _start_fetch(sc, jj, buf):
        off = pl.multiple_of(ebase + (sc * _SCH + jj) * _C, 8)
        ah = pltpu.async_copy(ea_hbm.at[pl.ds(off, _C)], a_bufs[buf],
                              a_sems[buf])
        gh = pltpu.async_copy(xa_hbm.at[row_v.at[jj]], g_bufs[buf],
                              g_sems[buf])
        return ah, gh

    def _sch(sc, _):
        pltpu.sync_copy(row_hbm.at[wid, sc], row_v)
        pltpu.sync_copy(col_hbm.at[wid, sc], col_v)
        pltpu.sync_copy(cidx_hbm.at[wid, sc], cidx_v)

        handles = {0: _start_fetch(sc, 0, 0)}
        scat = {}
        for jj in range(_SCH):
            cur = jj & 1
            if jj + 1 < _SCH:
                handles[jj + 1] = _start_fetch(sc, jj + 1, cur ^ 1)
            ah, gh = handles.pop(jj)
            ah.wait()
            gh.wait()
            if jj - 2 >= 0:
                scat.pop(jj - 2).wait()

            g_v, a_v = g_bufs[cur], a_bufs[cur]
            e_v = e_bufs[cur]

            def _row(r, _):
                for k in range(_H // 2 // _L):
                    sl = pl.ds(k * _L, _L)
                    aw = a_v[r, sl]
                    al = lax.bitcast_convert_type(aw << 16, jnp.float32)
                    ah2 = lax.bitcast_convert_type(aw & jnp.int32(-65536),
                                                   jnp.float32)
                    e_v[r, pl.ds(k * _L, _L)] = jnp.maximum(
                        g_v[r, pl.ds(k * _L, _L)] + al, jnp.float32(0.0))
                    e_v[r, pl.ds(_H // 2 + k * _L, _L)] = jnp.maximum(
                        g_v[r, pl.ds(_H // 2 + k * _L, _L)] + ah2,
                        jnp.float32(0.0))
                return 0
            lax.fori_loop(0, _C, _row, 0)

            g_lo = (jj * _C) // _L
            g_hi = ((jj + 1) * _C) // _L
            tail = ((jj + 1) * _C) % _L if jj == _SCH - 1 else 0

            def _hst(t, _):
                cvec = cidx_v[pl.ds(t * _L, _L)]
                for lane in range(_L):
                    cflat = cvec[lane]
                    r_i = cflat >> 9
                    base = pl.multiple_of((cflat & (_CW - 1)) & ~(_L - 1),
                                          _L)
                    inc = jnp.where(iota == (cflat & (_L - 1)), 1, 0
                                    ).astype(jnp.int32)
                    hist_v[r_i, pl.ds(base, _L)] = (
                        hist_v[r_i, pl.ds(base, _L)] + inc)
                return 0
            lax.fori_loop(g_lo, g_hi, _hst, 0)
            if tail:
                cvec = cidx_v[pl.ds(g_hi * _L, _L)]
                for lane in range(tail):
                    cflat = cvec[lane]
                    r_i = cflat >> 9
                    base = pl.multiple_of((cflat & (_CW - 1)) & ~(_L - 1),
                                          _L)
                    inc = jnp.where(iota == (cflat & (_L - 1)), 1, 0
                                    ).astype(jnp.int32)
                    hist_v[r_i, pl.ds(base, _L)] = (
                        hist_v[r_i, pl.ds(base, _L)] + inc)

            scat[jj] = pltpu.async_copy(e_v, seg_sh.at[col_v.at[jj]],
                                        s_sems[cur], add=True)
        for jj in sorted(scat):
            scat[jj].wait()
        return 0
    lax.fori_loop(0, _NSC, _sch, 0)

    pltpu.sync_copy(hist_v, outc_hbm.at[wid])

    plsc.subcore_barrier()
    for k in range(16):
        idx = s + _NS * k

        @pl.when(idx < _NZB)
        def _():
            off = pl.multiple_of(idx * _C, 8)
            pltpu.sync_copy(seg_sh.at[pl.ds(off, _C)],
                            outs_hbm.at[c, pl.ds(off, _C)])


def _combine_body(p_ref, c_ref, x_ref, u_ref, b_ref, w1b_ref, b1b_ref,
                  w2a_ref, b2a_ref, w2b_ref, b2b_ref, o_ref):
    ssum = p_ref[0]
    for i in range(1, _NC):
        ssum = ssum + p_ref[i]
    cvec = jnp.sum(c_ref[:, 0, 0, :], axis=0)[:_BN].astype(jnp.float32)
    inv = 1.0 / jnp.maximum(cvec, 1.0)
    cfrac = cvec * inv
    rr = lax.broadcasted_iota(jnp.int32, (_BN, _BN), 0)
    cc = lax.broadcasted_iota(jnp.int32, (_BN, _BN), 1)
    eye = (rr == cc).astype(jnp.float32)
    ones = jnp.ones((_BN, _H), jnp.float32)
    inv_b = jnp.dot(eye * inv[None, :], ones,
                    preferred_element_type=jnp.float32)
    cfrac_b = jnp.dot(eye * cfrac[None, :], ones,
                      preferred_element_type=jnp.float32)
    agg = (jnp.dot(ssum, w1b_ref[...], preferred_element_type=jnp.float32)
           * inv_b + cfrac_b * b1b_ref[...])
    bt = b_ref[0, 0, :]
    oh = (bt[:, None] == lax.broadcasted_iota(jnp.int32, (_BN, _NG), 1)
          ).astype(jnp.float32)
    uz = jnp.dot(u_ref[...], w2a_ref[2 * _H:, :],
                 preferred_element_type=jnp.float32)
    h = (jnp.dot(x_ref[...], w2a_ref[:_H, :],
                 preferred_element_type=jnp.float32)
         + jnp.dot(agg, w2a_ref[_H:2 * _H, :],
                   preferred_element_type=jnp.float32)
         + jnp.dot(oh, uz, preferred_element_type=jnp.float32)
         + b2a_ref[...])
    h = jnp.maximum(h, 0.0)
    o_ref[...] = (jnp.dot(h, w2b_ref[...], preferred_element_type=jnp.float32)
                  + b2b_ref[...])


_combine_call = pl.pallas_call(
    _combine_body,
    grid=(_N // _BN,),
    in_specs=[
        pl.BlockSpec((_NC, _BN, _H), lambda i: (0, i, 0)),
        pl.BlockSpec((_NW, 1, 1, _CW), lambda i: (0, i, 0, 0)),
        pl.BlockSpec((_BN, _DX), lambda i: (i, 0)),
        pl.BlockSpec((_NG, _H), lambda i: (0, 0)),
        pl.BlockSpec((1, 1, _BN), lambda i: (i, 0, 0)),
        pl.BlockSpec((_H, _H), lambda i: (0, 0)),
        pl.BlockSpec((1, _H), lambda i: (0, 0)),
        pl.BlockSpec((3 * _H, _H), lambda i: (0, 0)),
        pl.BlockSpec((1, _H), lambda i: (0, 0)),
        pl.BlockSpec((_H, _H), lambda i: (0, 0)),
        pl.BlockSpec((1, _H), lambda i: (0, 0)),
    ],
    out_specs=pl.BlockSpec((_BN, _H), lambda i: (i, 0)),
    out_shape=jax.ShapeDtypeStruct((_N, _H), jnp.float32),
)


def kernel(x, edge_index, edge_attr, u, batch,
           W1a, b1a, W1b, b1b, W2a, b2a, W2b, b2b):
    row = edge_index[0].reshape(_NW, _NSC, _SCH, _C)
    col = edge_index[1].reshape(_NW, _NSC, _SCH, _C)
    cidx = ((edge_index[1] // _BN) * _CW
            + edge_index[1] % _BN).reshape(_NW, _NSC, _SCH * _C)
    cidx = jnp.pad(cidx, ((0, 0), (0, 0), (0, _L)))
    ea, xa = _pre_call(edge_attr, W1a[_DX:], x, W1a[:_DX],
                       b1a.reshape(1, _H))
    seg, cnt = _sc_edge_call(xa, ea, row, col, cidx)
    cnt = cnt.reshape(_NW, _CR, 1, _CW)
    return _combine_call(seg, cnt, x, u, batch.reshape(_N // _BN, 1, _BN),
                         W1b, b1b.reshape(1, _H), W2a, b2a.reshape(1, _H),
                         W2b, b2b.reshape(1, _H))


# speedup vs baseline: 1.2244x; 1.2244x over previous
import functools

import jax
import jax.numpy as jnp
import numpy as np
from jax import lax
from jax.experimental import pallas as pl
from jax.experimental.pallas import tpu as pltpu
from jax.experimental.pallas import tpu_sc as plsc

_N = 10000
_E = 320000
_DX = 128
_DE = 16
_H = 128
_NG = 16
_L = 16
_NC = 1
_NS = 16
_NW = _NC * _NS
_EPT = _E // _NW
_C = 40
_NCH = _EPT // _C
_BN = 400
_CR = 25
_CW = 512
_SCH = 20
_NSC = _NCH // _SCH
_NZB = _N // _C


_EB = 12800
_NXB = _N // (_E // _EB)


def _bfpack(res):
    lo = lax.bitcast_convert_type(
        res[:, :_H // 2].astype(jnp.bfloat16), jnp.uint16).astype(jnp.uint32)
    hi = lax.bitcast_convert_type(
        res[:, _H // 2:].astype(jnp.bfloat16), jnp.uint16).astype(jnp.uint32)
    return lax.bitcast_convert_type(lo | (hi << 16), jnp.int32)


def _pre_body(a_ref, wa_ref, x_ref, wx_ref, b_ref, oe_ref, ox_ref):
    oe_ref[...] = _bfpack(jnp.dot(a_ref[...], wa_ref[...],
                                  preferred_element_type=jnp.float32))
    ox_ref[...] = (jnp.dot(x_ref[...], wx_ref[...],
                           preferred_element_type=jnp.float32)
                   + b_ref[...])


_pre_call = pl.pallas_call(
    _pre_body,
    grid=(_E // _EB,),
    in_specs=[
        pl.BlockSpec((_EB, _DE), lambda i: (i, 0)),
        pl.BlockSpec((_DE, _H), lambda i: (0, 0)),
        pl.BlockSpec((_NXB, _DX), lambda i: (i, 0)),
        pl.BlockSpec((_DX, _H), lambda i: (0, 0)),
        pl.BlockSpec((1, _H), lambda i: (0, 0)),
    ],
    out_specs=[
        pl.BlockSpec((_EB, _H // 2), lambda i: (i, 0)),
        pl.BlockSpec((_NXB, _H), lambda i: (i, 0)),
    ],
    out_shape=[
        jax.ShapeDtypeStruct((_E, _H // 2), jnp.int32),
        jax.ShapeDtypeStruct((_N, _H), jnp.float32),
    ],
)


@functools.partial(
    pl.kernel,
    out_type=(
        jax.ShapeDtypeStruct((_NC, _N, _H), jnp.float32),
        jax.ShapeDtypeStruct((_NW, _CR, _CW), jnp.int32),
    ),
    mesh=plsc.VectorSubcoreMesh(core_axis_name="c", subcore_axis_name="s",
                                num_cores=_NC, num_subcores=_NS),
    scratch_types=[
        pltpu.VMEM((_SCH, _C), jnp.int32),
        pltpu.VMEM((_SCH, _C), jnp.int32),
        pltpu.VMEM((_SCH * _C + _L,), jnp.int32),
        pltpu.VMEM((_C, _H), jnp.float32),
        pltpu.VMEM((_C, _H), jnp.float32),
        pltpu.VMEM((_C, _H), jnp.float32),
        pltpu.VMEM((_C, _H // 2), jnp.int32),
        pltpu.VMEM((_C, _H // 2), jnp.int32),
        pltpu.VMEM((_CR, _CW), jnp.int32),
        pltpu.VMEM_SHARED((_N, _H), jnp.float32),
        pltpu.SemaphoreType.DMA,
        pltpu.SemaphoreType.DMA,
        pltpu.SemaphoreType.DMA,
        pltpu.SemaphoreType.DMA,
        pltpu.SemaphoreType.DMA,
        pltpu.SemaphoreType.DMA,
        pltpu.SemaphoreType.DMA,
        pltpu.SemaphoreType.DMA,
    ],
)
def _sc_edge_call(xa_hbm, ea_hbm, row_hbm, col_hbm, cidx_hbm,
                  outs_hbm, outc_hbm, row_v, col_v, cidx_v,
                  g_b0, g_b1, g_b2, a_b0, a_b1, hist_v, seg_sh,
                  sem_g0, sem_g1, sem_g2, sem_a0, sem_a1,
                  sem_s0, sem_s1, sem_s2):
    c = lax.axis_index("c")
    s = lax.axis_index("s")
    wid = c * _NS + s

    def _zhrow(r, _):
        for k in range(_CW // _L):
            hist_v[r, pl.ds(k * _L, _L)] = jnp.zeros((_L,), jnp.int32)
        return 0
    lax.fori_loop(0, _CR, _zhrow, 0)

    def _zgrow(r, _):
        for k in range(_H // _L):
            g_b0[r, pl.ds(k * _L, _L)] = jnp.zeros((_L,), jnp.float32)
        return 0
    lax.fori_loop(0, _C, _zgrow, 0)

    iota = lax.iota(jnp.int32, _L)

    for k in range(16):
        idx = s + _NS * k

        @pl.when(idx < _NZB)
        def _():
            off = pl.multiple_of(idx * _C, 8)
            pltpu.sync_copy(g_b0, seg_sh.at[pl.ds(off, _C)])

    plsc.subcore_barrier()

    ebase = pl.multiple_of(wid * _EPT, 8)

    g_bufs = (g_b0, g_b1, g_b2)
    a_bufs = (a_b0, a_b1)
    g_sems = (sem_g0, sem_g1, sem_g2)
    a_sems = (sem_a0, sem_a1)
    s_sems = (sem_s0, sem_s1, sem_s2)

    def _start_fetch(sc, jj, abuf, gbuf):
        off = pl.multiple_of(ebase + (sc * _SCH + jj) * _C, 8)
        ah = pltpu.async_copy(ea_hbm.at[pl.ds(off, _C)], a_bufs[abuf],
                              a_sems[abuf])
        gh = pltpu.async_copy(xa_hbm.at[row_v.at[jj]], g_bufs[gbuf],
                              g_sems[gbuf])
        return ah, gh

    def _sch(sc, _):
        pltpu.sync_copy(row_hbm.at[wid, sc], row_v)
        pltpu.sync_copy(col_hbm.at[wid, sc], col_v)
        pltpu.sync_copy(cidx_hbm.at[wid, sc], cidx_v)

        handles = {0: _start_fetch(sc, 0, 0, 0)}
        scat = {}
        for jj in range(_SCH):
            cur = jj % 3
            if jj + 1 < _SCH:
                if jj - 2 >= 0:
                    scat.pop(jj - 2).wait()
                handles[jj + 1] = _start_fetch(sc, jj + 1, (jj + 1) & 1,
                                               (jj + 1) % 3)
            ah, gh = handles.pop(jj)
            ah.wait()
            gh.wait()

            a_v = a_bufs[jj & 1]
            e_v = g_bufs[cur]

            def _row(r, _):
                for k in range(_H // 2 // _L):
                    sl = pl.ds(k * _L, _L)
                    aw = a_v[r, sl]
                    al = lax.bitcast_convert_type(aw << 16, jnp.float32)
                    ah2 = lax.bitcast_convert_type(aw & jnp.int32(-65536),
                                                   jnp.float32)
                    slh = pl.ds(_H // 2 + k * _L, _L)
                    e_v[r, sl] = jnp.maximum(e_v[r, sl] + al,
                                             jnp.float32(0.0))
                    e_v[r, slh] = jnp.maximum(e_v[r, slh] + ah2,
                                              jnp.float32(0.0))
                return 0
            lax.fori_loop(0, _C, _row, 0)

            g_lo = (jj * _C) // _L
            g_hi = ((jj + 1) * _C) // _L
            tail = ((jj + 1) * _C) % _L if jj == _SCH - 1 else 0

            def _hst(t, _):
                cvec = cidx_v[pl.ds(t * _L, _L)]
                for lane in range(_L):
                    cflat = cvec[lane]
                    r_i = cflat >> 9
                    base = pl.multiple_of((cflat & (_CW - 1)) & ~(_L - 1),
                                          _L)
                    inc = jnp.where(iota == (cflat & (_L - 1)), 1, 0
                                    ).astype(jnp.int32)
                    hist_v[r_i, pl.ds(base, _L)] = (
                        hist_v[r_i, pl.ds(base, _L)] + inc)
                return 0
            lax.fori_loop(g_lo, g_hi, _hst, 0)
            if tail:
                cvec = cidx_v[pl.ds(g_hi * _L, _L)]
                for lane in range(tail):
                    cflat = cvec[lane]
                    r_i = cflat >> 9
                    base = pl.multiple_of((cflat & (_CW - 1)) & ~(_L - 1),
                                          _L)
                    inc = jnp.where(iota == (cflat & (_L - 1)), 1, 0
                                    ).astype(jnp.int32)
                    hist_v[r_i, pl.ds(base, _L)] = (
                        hist_v[r_i, pl.ds(base, _L)] + inc)

            scat[jj] = pltpu.async_copy(e_v, seg_sh.at[col_v.at[jj]],
                                        s_sems[cur], add=True)
        for jj in sorted(scat):
            scat[jj].wait()
        return 0
    lax.fori_loop(0, _NSC, _sch, 0)

    pltpu.sync_copy(hist_v, outc_hbm.at[wid])

    plsc.subcore_barrier()
    for k in range(16):
        idx = s + _NS * k

        @pl.when(idx < _NZB)
        def _():
            off = pl.multiple_of(idx * _C, 8)
            pltpu.sync_copy(seg_sh.at[pl.ds(off, _C)],
                            outs_hbm.at[c, pl.ds(off, _C)])


def _combine_body(p_ref, c_ref, x_ref, u_ref, b_ref, w1b_ref, b1b_ref,
                  w2a_ref, b2a_ref, w2b_ref, b2b_ref, o_ref):
    ssum = p_ref[0]
    for i in range(1, _NC):
        ssum = ssum + p_ref[i]
    cvec = jnp.sum(c_ref[:, 0, 0, :], axis=0)[:_BN].astype(jnp.float32)
    inv = 1.0 / jnp.maximum(cvec, 1.0)
    cfrac = cvec * inv
    rr = lax.broadcasted_iota(jnp.int32, (_BN, _BN), 0)
    cc = lax.broadcasted_iota(jnp.int32, (_BN, _BN), 1)
    eye = (rr == cc).astype(jnp.float32)
    ones = jnp.ones((_BN, _H), jnp.float32)
    inv_b = jnp.dot(eye * inv[None, :], ones,
                    preferred_element_type=jnp.float32)
    cfrac_b = jnp.dot(eye * cfrac[None, :], ones,
                      preferred_element_type=jnp.float32)
    agg = (jnp.dot(ssum, w1b_ref[...], preferred_element_type=jnp.float32)
           * inv_b + cfrac_b * b1b_ref[...])
    bt = b_ref[0, 0, :]
    oh = (bt[:, None] == lax.broadcasted_iota(jnp.int32, (_BN, _NG), 1)
          ).astype(jnp.float32)
    uz = jnp.dot(u_ref[...], w2a_ref[2 * _H:, :],
                 preferred_element_type=jnp.float32)
    h = (jnp.dot(x_ref[...], w2a_ref[:_H, :],
                 preferred_element_type=jnp.float32)
         + jnp.dot(agg, w2a_ref[_H:2 * _H, :],
                   preferred_element_type=jnp.float32)
         + jnp.dot(oh, uz, preferred_element_type=jnp.float32)
         + b2a_ref[...])
    h = jnp.maximum(h, 0.0)
    o_ref[...] = (jnp.dot(h, w2b_ref[...], preferred_element_type=jnp.float32)
                  + b2b_ref[...])


_combine_call = pl.pallas_call(
    _combine_body,
    grid=(_N // _BN,),
    in_specs=[
        pl.BlockSpec((_NC, _BN, _H), lambda i: (0, i, 0)),
        pl.BlockSpec((_NW, 1, 1, _CW), lambda i: (0, i, 0, 0)),
        pl.BlockSpec((_BN, _DX), lambda i: (i, 0)),
        pl.BlockSpec((_NG, _H), lambda i: (0, 0)),
        pl.BlockSpec((1, 1, _BN), lambda i: (i, 0, 0)),
        pl.BlockSpec((_H, _H), lambda i: (0, 0)),
        pl.BlockSpec((1, _H), lambda i: (0, 0)),
        pl.BlockSpec((3 * _H, _H), lambda i: (0, 0)),
        pl.BlockSpec((1, _H), lambda i: (0, 0)),
        pl.BlockSpec((_H, _H), lambda i: (0, 0)),
        pl.BlockSpec((1, _H), lambda i: (0, 0)),
    ],
    out_specs=pl.BlockSpec((_BN, _H), lambda i: (i, 0)),
    out_shape=jax.ShapeDtypeStruct((_N, _H), jnp.float32),
)


def kernel(x, edge_index, edge_attr, u, batch,
           W1a, b1a, W1b, b1b, W2a, b2a, W2b, b2b):
    row = edge_index[0].reshape(_NW, _NSC, _SCH, _C)
    col = edge_index[1].reshape(_NW, _NSC, _SCH, _C)
    cidx = ((edge_index[1] // _BN) * _CW
            + edge_index[1] % _BN).reshape(_NW, _NSC, _SCH * _C)
    cidx = jnp.pad(cidx, ((0, 0), (0, 0), (0, _L)))
    ea, xa = _pre_call(edge_attr, W1a[_DX:], x, W1a[:_DX],
                       b1a.reshape(1, _H))
    seg, cnt = _sc_edge_call(xa, ea, row, col, cidx)
    cnt = cnt.reshape(_NW, _CR, 1, _CW)
    return _combine_call(seg, cnt, x, u, batch.reshape(_N // _BN, 1, _BN),
                         W1b, b1b.reshape(1, _H), W2a, b2a.reshape(1, _H),
                         W2b, b2b.reshape(1, _H))


# speedup vs baseline: 1.3260x; 1.0830x over previous
import functools

import jax
import jax.numpy as jnp
import numpy as np
from jax import lax
from jax.experimental import pallas as pl
from jax.experimental.pallas import tpu as pltpu
from jax.experimental.pallas import tpu_sc as plsc

_N = 10000
_E = 320000
_DX = 128
_DE = 16
_H = 128
_NG = 16
_L = 16
_NC = 1
_NS = 16
_NW = _NC * _NS
_EPT = _E // _NW
_C = 40
_NCH = _EPT // _C
_BN = 400
_CR = 25
_CW = 512
_SCH = 20
_NSC = _NCH // _SCH
_NZB = _N // _C


_EB = 12800
_NXB = _N // (_E // _EB)


def _bfpack(res):
    lo = lax.bitcast_convert_type(
        res[:, :_H // 2].astype(jnp.bfloat16), jnp.uint16).astype(jnp.uint32)
    hi = lax.bitcast_convert_type(
        res[:, _H // 2:].astype(jnp.bfloat16), jnp.uint16).astype(jnp.uint32)
    return lax.bitcast_convert_type(lo | (hi << 16), jnp.int32)


def _pre_body(a_ref, wa_ref, x_ref, wx_ref, b_ref, oe_ref, ox_ref):
    oe_ref[...] = _bfpack(jnp.dot(a_ref[...], wa_ref[...],
                                  preferred_element_type=jnp.float32))
    ox_ref[...] = (jnp.dot(x_ref[...], wx_ref[...],
                           preferred_element_type=jnp.float32)
                   + b_ref[...])


_pre_call = pl.pallas_call(
    _pre_body,
    grid=(_E // _EB,),
    in_specs=[
        pl.BlockSpec((_EB, _DE), lambda i: (i, 0)),
        pl.BlockSpec((_DE, _H), lambda i: (0, 0)),
        pl.BlockSpec((_NXB, _DX), lambda i: (i, 0)),
        pl.BlockSpec((_DX, _H), lambda i: (0, 0)),
        pl.BlockSpec((1, _H), lambda i: (0, 0)),
    ],
    out_specs=[
        pl.BlockSpec((_EB, _H // 2), lambda i: (i, 0)),
        pl.BlockSpec((_NXB, _H), lambda i: (i, 0)),
    ],
    out_shape=[
        jax.ShapeDtypeStruct((_E, _H // 2), jnp.int32),
        jax.ShapeDtypeStruct((_N, _H), jnp.float32),
    ],
)


@functools.partial(
    pl.kernel,
    out_type=(
        jax.ShapeDtypeStruct((_NC, _N, _H), jnp.float32),
        jax.ShapeDtypeStruct((_NW, _CR, _CW), jnp.int32),
    ),
    mesh=plsc.VectorSubcoreMesh(core_axis_name="c", subcore_axis_name="s",
                                num_cores=_NC, num_subcores=_NS),
    scratch_types=[
        pltpu.VMEM((_SCH, _C), jnp.int32),
        pltpu.VMEM((_SCH, _C), jnp.int32),
        pltpu.VMEM((_SCH * _C + _L,), jnp.int32),
        pltpu.VMEM((_C, _H), jnp.float32),
        pltpu.VMEM((_C, _H), jnp.float32),
        pltpu.VMEM((_C, _H), jnp.float32),
        pltpu.VMEM((_C, _H // 2), jnp.int32),
        pltpu.VMEM((_C, _H // 2), jnp.int32),
        pltpu.VMEM((_CR, _CW), jnp.int32),
        pltpu.VMEM_SHARED((_N, _H), jnp.float32),
        pltpu.SemaphoreType.DMA,
        pltpu.SemaphoreType.DMA,
        pltpu.SemaphoreType.DMA,
        pltpu.SemaphoreType.DMA,
        pltpu.SemaphoreType.DMA,
        pltpu.SemaphoreType.DMA,
        pltpu.SemaphoreType.DMA,
        pltpu.SemaphoreType.DMA,
    ],
)
def _sc_edge_call(xa_hbm, ea_hbm, row_hbm, col_hbm, cidx_hbm,
                  outs_hbm, outc_hbm, row_v, col_v, cidx_v,
                  g_b0, g_b1, g_b2, a_b0, a_b1, hist_v, seg_sh,
                  sem_g0, sem_g1, sem_g2, sem_a0, sem_a1,
                  sem_s0, sem_s1, sem_s2):
    c = lax.axis_index("c")
    s = lax.axis_index("s")
    wid = c * _NS + s

    def _zhrow(r, _):
        for k in range(_CW // _L):
            hist_v[r, pl.ds(k * _L, _L)] = jnp.zeros((_L,), jnp.int32)
        return 0
    lax.fori_loop(0, _CR, _zhrow, 0)

    def _zgrow(r, _):
        for k in range(_H // _L):
            g_b0[r, pl.ds(k * _L, _L)] = jnp.zeros((_L,), jnp.float32)
        return 0
    lax.fori_loop(0, _C, _zgrow, 0)

    iota = lax.iota(jnp.int32, _L)

    for k in range(16):
        idx = s + _NS * k

        @pl.when(idx < _NZB)
        def _():
            off = pl.multiple_of(idx * _C, 8)
            pltpu.sync_copy(g_b0, seg_sh.at[pl.ds(off, _C)])

    plsc.subcore_barrier()

    ebase = pl.multiple_of(wid * _EPT, 8)

    g_bufs = (g_b0, g_b1, g_b2)
    a_bufs = (a_b0, a_b1)
    g_sems = (sem_g0, sem_g1, sem_g2)
    a_sems = (sem_a0, sem_a1)
    s_sems = (sem_s0, sem_s1, sem_s2)

    def _start_fetch(sc, jj, abuf, gbuf):
        off = pl.multiple_of(ebase + (sc * _SCH + jj) * _C, 8)
        ah = pltpu.async_copy(ea_hbm.at[pl.ds(off, _C)], a_bufs[abuf],
                              a_sems[abuf])
        gh = pltpu.async_copy(xa_hbm.at[row_v.at[jj]], g_bufs[gbuf],
                              g_sems[gbuf])
        return ah, gh

    def _sch(sc, _):
        pltpu.sync_copy(row_hbm.at[wid, sc], row_v)
        pltpu.sync_copy(col_hbm.at[wid, sc], col_v)
        pltpu.sync_copy(cidx_hbm.at[wid, sc], cidx_v)

        handles = {0: _start_fetch(sc, 0, 0, 0)}
        scat = {}
        for jj in range(_SCH):
            cur = jj % 3
            if jj + 1 < _SCH:
                if jj - 2 >= 0:
                    scat.pop(jj - 2).wait()
                handles[jj + 1] = _start_fetch(sc, jj + 1, (jj + 1) & 1,
                                               (jj + 1) % 3)
            g_lo = (jj * _C) // _L
            g_hi = ((jj + 1) * _C) // _L
            tail = ((jj + 1) * _C) % _L if jj == _SCH - 1 else 0

            def _hst(t, _):
                cvec = cidx_v[pl.ds(t * _L, _L)]
                for lane in range(_L):
                    cflat = cvec[lane]
                    r_i = cflat >> 9
                    base = pl.multiple_of((cflat & (_CW - 1)) & ~(_L - 1),
                                          _L)
                    inc = jnp.where(iota == (cflat & (_L - 1)), 1, 0
                                    ).astype(jnp.int32)
                    hist_v[r_i, pl.ds(base, _L)] = (
                        hist_v[r_i, pl.ds(base, _L)] + inc)
                return 0
            lax.fori_loop(g_lo, g_hi, _hst, 0)
            if tail:
                cvec = cidx_v[pl.ds(g_hi * _L, _L)]
                for lane in range(tail):
                    cflat = cvec[lane]
                    r_i = cflat >> 9
                    base = pl.multiple_of((cflat & (_CW - 1)) & ~(_L - 1),
                                          _L)
                    inc = jnp.where(iota == (cflat & (_L - 1)), 1, 0
                                    ).astype(jnp.int32)
                    hist_v[r_i, pl.ds(base, _L)] = (
                        hist_v[r_i, pl.ds(base, _L)] + inc)

            ah, gh = handles.pop(jj)
            ah.wait()
            gh.wait()

            a_v = a_bufs[jj & 1]
            e_v = g_bufs[cur]

            def _row(r, _):
                for k in range(_H // 2 // _L):
                    sl = pl.ds(k * _L, _L)
                    aw = a_v[r, sl]
                    al = lax.bitcast_convert_type(aw << 16, jnp.float32)
                    ah2 = lax.bitcast_convert_type(aw & jnp.int32(-65536),
                                                   jnp.float32)
                    slh = pl.ds(_H // 2 + k * _L, _L)
                    e_v[r, sl] = jnp.maximum(e_v[r, sl] + al,
                                             jnp.float32(0.0))
                    e_v[r, slh] = jnp.maximum(e_v[r, slh] + ah2,
                                              jnp.float32(0.0))
                return 0
            lax.fori_loop(0, _C, _row, 0)

            scat[jj] = pltpu.async_copy(e_v, seg_sh.at[col_v.at[jj]],
                                        s_sems[cur], add=True)
        for jj in sorted(scat):
            scat[jj].wait()
        return 0
    lax.fori_loop(0, _NSC, _sch, 0)

    pltpu.sync_copy(hist_v, outc_hbm.at[wid])

    plsc.subcore_barrier()
    for k in range(16):
        idx = s + _NS * k

        @pl.when(idx < _NZB)
        def _():
            off = pl.multiple_of(idx * _C, 8)
            pltpu.sync_copy(seg_sh.at[pl.ds(off, _C)],
                            outs_hbm.at[c, pl.ds(off, _C)])


def _combine_body(p_ref, c_ref, x_ref, u_ref, b_ref, w1b_ref, b1b_ref,
                  w2a_ref, b2a_ref, w2b_ref, b2b_ref, o_ref):
    ssum = p_ref[0]
    for i in range(1, _NC):
        ssum = ssum + p_ref[i]
    cvec = jnp.sum(c_ref[:, 0, 0, :], axis=0)[:_BN].astype(jnp.float32)
    inv = 1.0 / jnp.maximum(cvec, 1.0)
    cfrac = cvec * inv
    rr = lax.broadcasted_iota(jnp.int32, (_BN, _BN), 0)
    cc = lax.broadcasted_iota(jnp.int32, (_BN, _BN), 1)
    eye = (rr == cc).astype(jnp.float32)
    ones = jnp.ones((_BN, _H), jnp.float32)
    inv_b = jnp.dot(eye * inv[None, :], ones,
                    preferred_element_type=jnp.float32)
    cfrac_b = jnp.dot(eye * cfrac[None, :], ones,
                      preferred_element_type=jnp.float32)
    agg = (jnp.dot(ssum, w1b_ref[...], preferred_element_type=jnp.float32)
           * inv_b + cfrac_b * b1b_ref[...])
    bt = b_ref[0, 0, :]
    oh = (bt[:, None] == lax.broadcasted_iota(jnp.int32, (_BN, _NG), 1)
          ).astype(jnp.float32)
    uz = jnp.dot(u_ref[...], w2a_ref[2 * _H:, :],
                 preferred_element_type=jnp.float32)
    h = (jnp.dot(x_ref[...], w2a_ref[:_H, :],
                 preferred_element_type=jnp.float32)
         + jnp.dot(agg, w2a_ref[_H:2 * _H, :],
                   preferred_element_type=jnp.float32)
         + jnp.dot(oh, uz, preferred_element_type=jnp.float32)
         + b2a_ref[...])
    h = jnp.maximum(h, 0.0)
    o_ref[...] = (jnp.dot(h, w2b_ref[...], preferred_element_type=jnp.float32)
                  + b2b_ref[...])


_combine_call = pl.pallas_call(
    _combine_body,
    grid=(_N // _BN,),
    in_specs=[
        pl.BlockSpec((_NC, _BN, _H), lambda i: (0, i, 0)),
        pl.BlockSpec((_NW, 1, 1, _CW), lambda i: (0, i, 0, 0)),
        pl.BlockSpec((_BN, _DX), lambda i: (i, 0)),
        pl.BlockSpec((_NG, _H), lambda i: (0, 0)),
        pl.BlockSpec((1, 1, _BN), lambda i: (i, 0, 0)),
        pl.BlockSpec((_H, _H), lambda i: (0, 0)),
        pl.BlockSpec((1, _H), lambda i: (0, 0)),
        pl.BlockSpec((3 * _H, _H), lambda i: (0, 0)),
        pl.BlockSpec((1, _H), lambda i: (0, 0)),
        pl.BlockSpec((_H, _H), lambda i: (0, 0)),
        pl.BlockSpec((1, _H), lambda i: (0, 0)),
    ],
    out_specs=pl.BlockSpec((_BN, _H), lambda i: (i, 0)),
    out_shape=jax.ShapeDtypeStruct((_N, _H), jnp.float32),
)


def kernel(x, edge_index, edge_attr, u, batch,
           W1a, b1a, W1b, b1b, W2a, b2a, W2b, b2b):
    row = edge_index[0].reshape(_NW, _NSC, _SCH, _C)
    col = edge_index[1].reshape(_NW, _NSC, _SCH, _C)
    cidx = ((edge_index[1] // _BN) * _CW
            + edge_index[1] % _BN).reshape(_NW, _NSC, _SCH * _C)
    cidx = jnp.pad(cidx, ((0, 0), (0, 0), (0, _L)))
    ea, xa = _pre_call(edge_attr, W1a[_DX:], x, W1a[:_DX],
                       b1a.reshape(1, _H))
    seg, cnt = _sc_edge_call(xa, ea, row, col, cidx)
    cnt = cnt.reshape(_NW, _CR, 1, _CW)
    return _combine_call(seg, cnt, x, u, batch.reshape(_N // _BN, 1, _BN),
                         W1b, b1b.reshape(1, _H), W2a, b2a.reshape(1, _H),
                         W2b, b2b.reshape(1, _H))
